# Initial kernel scaffold; baseline (speedup 1.0000x reference)
import jax
import jax.numpy as jnp
from jax.experimental import pallas as pl


def kernel(h, pos, edge_index, msg_W1, msg_b1, msg_W2, msg_b2, coord_W1, coord_b1, coord_W2, coord_b2, node_W1, node_b1, node_W2, node_b2):
    raise NotImplementedError("write your pallas kernel here")


# speedup vs baseline: 2.6572x; 2.6572x over previous
import dataclasses
import functools

import jax
import jax.numpy as jnp
from jax import lax
from jax.experimental import pallas as pl
from jax.experimental.pallas import tpu as pltpu
from jax.experimental.pallas import tpu_sc as plsc

N = 10000
E = 320000
D = 128
P16 = 16
CLAMP = 10.0

NC = 2
NS = 16
NW = NC * NS
EPT = E // NW
CHUNK = 80
NCHUNK = EPT // CHUNK

EPC = E // NC
NPTA = 624


def _silu(x):
    return x * jax.nn.sigmoid(x)


def _preproj_body(h_ref, w1s_ref, w1d_ref, b1_ref, a_ref, b_ref):
    h = h_ref[...]
    a_ref[...] = (
        jnp.dot(h, w1s_ref[...], preferred_element_type=jnp.float32,
                precision=lax.Precision.HIGHEST)
        + b1_ref[...]
    )
    b_ref[...] = jnp.dot(h, w1d_ref[...], preferred_element_type=jnp.float32,
                         precision=lax.Precision.HIGHEST)


def _preproj(h, w1s, w1d, b1):
    blk = 1000
    grid = (N // blk,)
    return pl.pallas_call(
        _preproj_body,
        grid=grid,
        in_specs=[
            pl.BlockSpec((blk, D), lambda i: (i, 0)),
            pl.BlockSpec((D, D), lambda i: (0, 0)),
            pl.BlockSpec((D, D), lambda i: (0, 0)),
            pl.BlockSpec((1, D), lambda i: (0, 0)),
        ],
        out_specs=[
            pl.BlockSpec((blk, D), lambda i: (i, 0)),
            pl.BlockSpec((blk, D), lambda i: (i, 0)),
        ],
        out_shape=[
            jax.ShapeDtypeStruct((N, D), jnp.float32),
            jax.ShapeDtypeStruct((N, D), jnp.float32),
        ],
    )(h, w1s, w1d, b1)


def _sc_cp():
    cp = pltpu.CompilerParams()
    if "needs_layout_passes" in pltpu.CompilerParams.__dataclass_fields__:
        cp = dataclasses.replace(cp, needs_layout_passes=False)
    return cp


def _sc_gather(a, b, px, py, pz, src, dst):
    mesh = plsc.VectorSubcoreMesh(core_axis_name="c", subcore_axis_name="s")
    L = 16
    NG = CHUNK // L

    @functools.partial(
        pl.kernel,
        mesh=mesh,
        compiler_params=_sc_cp(),
        out_type=[
            jax.ShapeDtypeStruct((E, D), jnp.float32),
            jax.ShapeDtypeStruct((E, D), jnp.float32),
            jax.ShapeDtypeStruct((E, P16), jnp.float32),
        ],
        scratch_types=[
            pltpu.VMEM((CHUNK,), jnp.int32),
            pltpu.VMEM((CHUNK,), jnp.int32),
            pltpu.VMEM((CHUNK, D), jnp.float32),
            pltpu.VMEM((CHUNK, D), jnp.float32),
            pltpu.VMEM((CHUNK, P16), jnp.float32),
            pltpu.VMEM((N,), jnp.float32),
            pltpu.VMEM((N,), jnp.float32),
            pltpu.VMEM((N,), jnp.float32),
            pltpu.SemaphoreType.DMA,
        ],
    )
    def k(a_hbm, b_hbm, px_hbm, py_hbm, pz_hbm, src_hbm, dst_hbm,
          ag_hbm, bg_hbm, geo_hbm,
          sidx_v, didx_v, a_v, b_v, geo_v, px_v, py_v, pz_v, sem):
        wid = lax.axis_index("s") * NC + lax.axis_index("c")
        base = wid * EPT

        pltpu.sync_copy(px_hbm, px_v)
        pltpu.sync_copy(py_hbm, py_v)
        pltpu.sync_copy(pz_hbm, pz_v)

        zeros16 = jnp.zeros((L,), jnp.float32)

        @pl.loop(0, NG)
        def _(g):
            @pl.loop(0, L)
            def _(i):
                geo_v[g * L + i, :] = zeros16

        @pl.loop(0, NCHUNK)
        def _(c):
            off = base + c * CHUNK
            pltpu.sync_copy(src_hbm.at[pl.ds(off, CHUNK)], sidx_v)
            pltpu.sync_copy(dst_hbm.at[pl.ds(off, CHUNK)], didx_v)
            cp_a = pltpu.async_copy(a_hbm.at[sidx_v], a_v, sem)
            cp_b = pltpu.async_copy(b_hbm.at[didx_v], b_v, sem)
            for g in range(NG):
                si = sidx_v[pl.ds(g * L, L)]
                di = didx_v[pl.ds(g * L, L)]
                dx = plsc.load_gather(px_v, [si]) - plsc.load_gather(px_v, [di])
                dy = plsc.load_gather(py_v, [si]) - plsc.load_gather(py_v, [di])
                dz = plsc.load_gather(pz_v, [si]) - plsc.load_gather(pz_v, [di])
                d2 = dx * dx + dy * dy + dz * dz
                rows = g * L + lax.iota(jnp.int32, L)
                plsc.store_scatter(geo_v, [rows, jnp.full((L,), 0, jnp.int32)], dx)
                plsc.store_scatter(geo_v, [rows, jnp.full((L,), 1, jnp.int32)], dy)
                plsc.store_scatter(geo_v, [rows, jnp.full((L,), 2, jnp.int32)], dz)
                plsc.store_scatter(geo_v, [rows, jnp.full((L,), 3, jnp.int32)], d2)
            cp_a.wait()
            cp_b.wait()
            pltpu.sync_copy(a_v, ag_hbm.at[pl.ds(off, CHUNK)])
            pltpu.sync_copy(b_v, bg_hbm.at[pl.ds(off, CHUNK)])
            pltpu.sync_copy(geo_v, geo_hbm.at[pl.ds(off, CHUNK)])

    return k(a, b, px, py, pz, src, dst)


def _edge_body(ag_ref, bg_ref, geo_ref,
               w1dist_ref, w2_ref, b2_ref, cw1_ref, cb1_ref, cw2_ref,
               cb2_ref, msg_ref, dcw_ref):
    geo = geo_ref[...]
    d2 = geo[:, 3:4]
    pre = ag_ref[...] + bg_ref[...] + d2 * w1dist_ref[...]
    m = _silu(pre)
    msg = _silu(
        jnp.dot(m, w2_ref[...], preferred_element_type=jnp.float32,
                precision=lax.Precision.HIGHEST) + b2_ref[...])
    u = _silu(
        jnp.dot(msg, cw1_ref[...], preferred_element_type=jnp.float32,
                precision=lax.Precision.HIGHEST) + cb1_ref[...])
    cwv = jnp.dot(u, cw2_ref[...], preferred_element_type=jnp.float32,
                  precision=lax.Precision.HIGHEST) + cb2_ref[...]
    cw = jnp.clip(cwv[:, :1], -CLAMP, CLAMP)
    msg_ref[...] = msg
    lane = lax.broadcasted_iota(jnp.int32, geo.shape, 1)
    dcw_ref[...] = jnp.where(lane < 3, geo * cw, 0.0)


def _edge_mlp(ag, bg, geo, w1dist, w2, b2, cw1, cb1, cw2p, cb2p):
    blk = 2000
    grid = (E // blk,)
    wspec = lambda shape: pl.BlockSpec(shape, lambda i: (0, 0))
    return pl.pallas_call(
        _edge_body,
        grid=grid,
        in_specs=[
            pl.BlockSpec((blk, D), lambda i: (i, 0)),
            pl.BlockSpec((blk, D), lambda i: (i, 0)),
            pl.BlockSpec((blk, P16), lambda i: (i, 0)),
            wspec((1, D)),
            wspec((D, D)),
            wspec((1, D)),
            wspec((D, D)),
            wspec((1, D)),
            wspec((D, 8)),
            wspec((1, 8)),
        ],
        out_specs=[
            pl.BlockSpec((blk, D), lambda i: (i, 0)),
            pl.BlockSpec((blk, P16), lambda i: (i, 0)),
        ],
        out_shape=[
            jax.ShapeDtypeStruct((E, D), jnp.float32),
            jax.ShapeDtypeStruct((E, P16), jnp.float32),
        ],
    )(ag, bg, geo, w1dist, w2, b2, cw1, cb1, cw2p, cb2p)


def _sc_scatter(msg, dcw, dst, z128):
    mesh = plsc.VectorSubcoreMesh(core_axis_name="c", subcore_axis_name="s")
    L = 16

    @functools.partial(
        pl.kernel,
        mesh=mesh,
        compiler_params=_sc_cp(),
        out_type=[
            jax.ShapeDtypeStruct((NC, N, D), jnp.float32),
            jax.ShapeDtypeStruct((NC, N, D), jnp.float32),
        ],
        scratch_types=[
            pltpu.VMEM_SHARED((N, D), jnp.float32),
            pltpu.VMEM((CHUNK,), jnp.int32),
            pltpu.VMEM((CHUNK, D), jnp.float32),
            pltpu.VMEM((CHUNK, P16), jnp.float32),
            pltpu.VMEM((CHUNK, D), jnp.float32),
        ],
    )
    def k(msg_hbm, dcw_hbm, dst_hbm, z128_hbm,
          outa_hbm, outb_hbm, acc, didx_v, m_v, d_v, d128_v):
        cid = lax.axis_index("c")
        sid = lax.axis_index("s")
        base = cid * EPC + sid * EPT
        rows = pl.ds(sid * NPTA, NPTA)
        tail = pl.ds(NS * NPTA, N - NS * NPTA)
        zeros16 = jnp.zeros((L,), jnp.float32)

        @pl.loop(0, CHUNK)
        def _(i):
            for j in range(D // L):
                d128_v[i, pl.ds(j * L, L)] = zeros16

        def zero_acc():
            pltpu.sync_copy(z128_hbm.at[rows], acc.at[rows])

            @pl.when(sid == NS - 1)
            def _():
                pltpu.sync_copy(z128_hbm.at[tail], acc.at[tail])

        def drain(out_hbm):
            pltpu.sync_copy(acc.at[rows], out_hbm.at[cid].at[rows])

            @pl.when(sid == NS - 1)
            def _():
                pltpu.sync_copy(acc.at[tail], out_hbm.at[cid].at[tail])

        zero_acc()
        plsc.subcore_barrier()

        @pl.loop(0, NCHUNK)
        def _(c):
            off = base + c * CHUNK
            pltpu.sync_copy(dst_hbm.at[pl.ds(off, CHUNK)], didx_v)
            pltpu.sync_copy(msg_hbm.at[pl.ds(off, CHUNK)], m_v)
            pltpu.sync_copy(m_v, acc.at[didx_v], add=True)

        plsc.subcore_barrier()
        drain(outa_hbm)

        zero_acc()
        plsc.subcore_barrier()

        @pl.loop(0, NCHUNK)
        def _(c):
            off = base + c * CHUNK
            pltpu.sync_copy(dst_hbm.at[pl.ds(off, CHUNK)], didx_v)
            pltpu.sync_copy(dcw_hbm.at[pl.ds(off, CHUNK)], d_v)

            @pl.loop(0, CHUNK)
            def _(i):
                d128_v[i, pl.ds(0, L)] = d_v[i, pl.ds(0, L)]

            pltpu.sync_copy(d128_v, acc.at[didx_v], add=True)

        plsc.subcore_barrier()
        drain(outb_hbm)

    return k(msg, dcw, dst, z128)


def _node_body(h_ref, p16_ref, pa0_ref, pa1_ref, pb0_ref, pb1_ref,
               nw1h_ref, nw1m_ref, nb1_ref, nw2_ref, nb2_ref,
               hn_ref, pn_ref):
    h = h_ref[...]
    magg = pa0_ref[...] + pa1_ref[...]
    t = _silu(
        jnp.dot(h, nw1h_ref[...], preferred_element_type=jnp.float32,
                precision=lax.Precision.HIGHEST)
        + jnp.dot(magg, nw1m_ref[...], preferred_element_type=jnp.float32,
                  precision=lax.Precision.HIGHEST)
        + nb1_ref[...])
    hn_ref[...] = h + jnp.dot(
        t, nw2_ref[...], preferred_element_type=jnp.float32,
        precision=lax.Precision.HIGHEST) + nb2_ref[...]
    pn_ref[...] = p16_ref[...] + pb0_ref[:, :P16] + pb1_ref[:, :P16]


def _node_mlp(h, pos16, pa0, pa1, pb0, pb1, nw1h, nw1m, nb1, nw2, nb2):
    blk = 1000
    grid = (N // blk,)
    wspec = lambda shape: pl.BlockSpec(shape, lambda i: (0, 0))
    return pl.pallas_call(
        _node_body,
        grid=grid,
        in_specs=[
            pl.BlockSpec((blk, D), lambda i: (i, 0)),
            pl.BlockSpec((blk, P16), lambda i: (i, 0)),
            pl.BlockSpec((blk, D), lambda i: (i, 0)),
            pl.BlockSpec((blk, D), lambda i: (i, 0)),
            pl.BlockSpec((blk, D), lambda i: (i, 0)),
            pl.BlockSpec((blk, D), lambda i: (i, 0)),
            wspec((D, D)),
            wspec((D, D)),
            wspec((1, D)),
            wspec((D, D)),
            wspec((1, D)),
        ],
        out_specs=[
            pl.BlockSpec((blk, D), lambda i: (i, 0)),
            pl.BlockSpec((blk, P16), lambda i: (i, 0)),
        ],
        out_shape=[
            jax.ShapeDtypeStruct((N, D), jnp.float32),
            jax.ShapeDtypeStruct((N, P16), jnp.float32),
        ],
    )(h, pos16, pa0, pa1, pb0, pb1, nw1h, nw1m, nb1, nw2, nb2)


def kernel(h, pos, edge_index, msg_W1, msg_b1, msg_W2, msg_b2,
           coord_W1, coord_b1, coord_W2, coord_b2,
           node_W1, node_b1, node_W2, node_b2):
    src = edge_index[0]
    dst = edge_index[1]

    w1s = msg_W1[:D]
    w1d = msg_W1[D:2 * D]
    w1dist = msg_W1[2 * D].reshape(1, D)
    b1 = msg_b1.reshape(1, D)
    b2 = msg_b2.reshape(1, D)
    cb1 = coord_b1.reshape(1, D)
    cw2p = jnp.pad(coord_W2, ((0, 0), (0, 7)))
    cb2p = jnp.pad(coord_b2.reshape(1, 1), ((0, 0), (0, 7)))
    nw1h = node_W1[:D]
    nw1m = node_W1[D:]
    nb1 = node_b1.reshape(1, D)
    nb2 = node_b2.reshape(1, D)

    pos16 = jnp.pad(pos, ((0, 0), (0, P16 - 3)))
    z128 = jnp.zeros((N, D), jnp.float32)

    a, b = _preproj(h, w1s, w1d, b1)
    px = jnp.asarray(pos[:, 0])
    py = jnp.asarray(pos[:, 1])
    pz = jnp.asarray(pos[:, 2])
    ag, bg, geo = _sc_gather(a, b, px, py, pz, src, dst)
    msg, dcw = _edge_mlp(ag, bg, geo, w1dist, msg_W2, b2,
                         coord_W1, cb1, cw2p, cb2p)
    parts_a, parts_b = _sc_scatter(msg, dcw, dst, z128)
    h_new, pos16_new = _node_mlp(h, pos16, parts_a[0], parts_a[1],
                                 parts_b[0], parts_b[1],
                                 nw1h, nw1m, nb1, node_W2, nb2)
    return (h_new, pos16_new[:, :3])


# speedup vs baseline: 2.9784x; 1.1209x over previous
import dataclasses
import functools

import jax
import jax.numpy as jnp
from jax import lax
from jax.experimental import pallas as pl
from jax.experimental.pallas import tpu as pltpu
from jax.experimental.pallas import tpu_sc as plsc

N = 10000
E = 320000
D = 128
P16 = 16
CLAMP = 10.0

NC = 2
NS = 16
NW = NC * NS
EPT = E // NW
CHUNK = 80
NCHUNK = EPT // CHUNK

EPC = E // NC
NPTA = 624


def _silu(x):
    return x * jax.nn.sigmoid(x)


def _preproj_body(h_ref, w1s_ref, w1d_ref, b1_ref, a_ref, b_ref):
    h = h_ref[...]
    a_ref[...] = (
        jnp.dot(h, w1s_ref[...], preferred_element_type=jnp.float32,
                precision=lax.Precision.HIGHEST)
        + b1_ref[...]
    )
    b_ref[...] = jnp.dot(h, w1d_ref[...], preferred_element_type=jnp.float32,
                         precision=lax.Precision.HIGHEST)


def _preproj(h, w1s, w1d, b1):
    blk = 1000
    grid = (N // blk,)
    return pl.pallas_call(
        _preproj_body,
        grid=grid,
        in_specs=[
            pl.BlockSpec((blk, D), lambda i: (i, 0)),
            pl.BlockSpec((D, D), lambda i: (0, 0)),
            pl.BlockSpec((D, D), lambda i: (0, 0)),
            pl.BlockSpec((1, D), lambda i: (0, 0)),
        ],
        out_specs=[
            pl.BlockSpec((blk, D), lambda i: (i, 0)),
            pl.BlockSpec((blk, D), lambda i: (i, 0)),
        ],
        out_shape=[
            jax.ShapeDtypeStruct((N, D), jnp.float32),
            jax.ShapeDtypeStruct((N, D), jnp.float32),
        ],
    )(h, w1s, w1d, b1)


def _sc_cp():
    cp = pltpu.CompilerParams()
    if "needs_layout_passes" in pltpu.CompilerParams.__dataclass_fields__:
        cp = dataclasses.replace(cp, needs_layout_passes=False)
    return cp


NBUF = 3
RING = (NCHUNK // NBUF) * NBUF


def _sc_gather(a, b, px, py, pz, src, dst):
    mesh = plsc.VectorSubcoreMesh(core_axis_name="c", subcore_axis_name="s")
    L = 16
    NG = CHUNK // L

    scratch = (
        [pltpu.VMEM((CHUNK,), jnp.int32) for _ in range(2 * NBUF)]
        + [pltpu.VMEM((CHUNK, D), jnp.float32) for _ in range(2 * NBUF)]
        + [pltpu.VMEM((CHUNK, P16), jnp.float32) for _ in range(NBUF)]
        + [pltpu.VMEM((N,), jnp.float32) for _ in range(3)]
        + [pltpu.SemaphoreType.DMA for _ in range(3 * NBUF)]
    )

    @functools.partial(
        pl.kernel,
        mesh=mesh,
        compiler_params=_sc_cp(),
        out_type=[
            jax.ShapeDtypeStruct((E, D), jnp.float32),
            jax.ShapeDtypeStruct((E, D), jnp.float32),
            jax.ShapeDtypeStruct((E, P16), jnp.float32),
        ],
        scratch_types=scratch,
    )
    def k(a_hbm, b_hbm, px_hbm, py_hbm, pz_hbm, src_hbm, dst_hbm,
          ag_hbm, bg_hbm, geo_hbm, *sc):
        sidx = sc[0:NBUF]
        didx = sc[NBUF:2 * NBUF]
        av = sc[2 * NBUF:3 * NBUF]
        bv = sc[3 * NBUF:4 * NBUF]
        geov = sc[4 * NBUF:5 * NBUF]
        px_v, py_v, pz_v = sc[5 * NBUF:5 * NBUF + 3]
        sem_i = sc[5 * NBUF + 3:5 * NBUF + 3 + NBUF]
        sem_g = sc[5 * NBUF + 3 + NBUF:5 * NBUF + 3 + 2 * NBUF]
        sem_o = sc[5 * NBUF + 3 + 2 * NBUF:5 * NBUF + 3 + 3 * NBUF]

        wid = lax.axis_index("s") * NC + lax.axis_index("c")
        base = wid * EPT
        zeros16 = jnp.zeros((L,), jnp.float32)

        pltpu.sync_copy(px_hbm, px_v)
        pltpu.sync_copy(py_hbm, py_v)
        pltpu.sync_copy(pz_hbm, pz_v)

        for bslot in range(NBUF):
            @pl.loop(0, CHUNK)
            def _(i, _b=bslot):
                geov[_b][i, :] = zeros16

        def idx_cp(bslot, c, which):
            off = base + c * CHUNK
            hbm = src_hbm if which == 0 else dst_hbm
            buf = sidx[bslot] if which == 0 else didx[bslot]
            return pltpu.make_async_copy(hbm.at[pl.ds(off, CHUNK)], buf,
                                         sem_i[bslot])

        def gat_cp(bslot, which):
            if which == 0:
                return pltpu.make_async_copy(a_hbm.at[sidx[bslot]], av[bslot],
                                             sem_g[bslot])
            return pltpu.make_async_copy(b_hbm.at[didx[bslot]], bv[bslot],
                                         sem_g[bslot])

        def out_cp(bslot, c, which):
            off = base + c * CHUNK
            src_v, dst_h = [(av[bslot], ag_hbm), (bv[bslot], bg_hbm),
                            (geov[bslot], geo_hbm)][which]
            return pltpu.make_async_copy(src_v, dst_h.at[pl.ds(off, CHUNK)],
                                         sem_o[bslot])

        def geo_compute(bslot):
            for g in range(NG):
                si = sidx[bslot][pl.ds(g * L, L)]
                di = didx[bslot][pl.ds(g * L, L)]
                dx = plsc.load_gather(px_v, [si]) - plsc.load_gather(px_v, [di])
                dy = plsc.load_gather(py_v, [si]) - plsc.load_gather(py_v, [di])
                dz = plsc.load_gather(pz_v, [si]) - plsc.load_gather(pz_v, [di])
                d2 = dx * dx + dy * dy + dz * dz
                rows = g * L + lax.iota(jnp.int32, L)
                plsc.store_scatter(geov[bslot],
                                   [rows, jnp.full((L,), 0, jnp.int32)], dx)
                plsc.store_scatter(geov[bslot],
                                   [rows, jnp.full((L,), 1, jnp.int32)], dy)
                plsc.store_scatter(geov[bslot],
                                   [rows, jnp.full((L,), 2, jnp.int32)], dz)
                plsc.store_scatter(geov[bslot],
                                   [rows, jnp.full((L,), 3, jnp.int32)], d2)

        for c in range(RING, NCHUNK):
            idx_cp(0, c, 0).start()
            idx_cp(0, c, 1).start()
            idx_cp(0, c, 0).wait()
            idx_cp(0, c, 1).wait()
            gat_cp(0, 0).start()
            gat_cp(0, 1).start()
            geo_compute(0)
            gat_cp(0, 0).wait()
            gat_cp(0, 1).wait()
            out_cp(0, c, 0).start()
            out_cp(0, c, 1).start()
            out_cp(0, c, 2).start()
            out_cp(0, c, 0).wait()
            out_cp(0, c, 1).wait()
            out_cp(0, c, 2).wait()

        for bslot in range(NBUF):
            idx_cp(bslot, bslot, 0).start()
            idx_cp(bslot, bslot, 1).start()

        @pl.loop(0, RING // NBUF)
        def _(s):
            for bslot in range(NBUF):
                c = s * NBUF + bslot

                @pl.when(s > 0)
                def _(_b=bslot, _c=c):
                    out_cp(_b, _c - NBUF, 0).wait()
                    out_cp(_b, _c - NBUF, 1).wait()
                    out_cp(_b, _c - NBUF, 2).wait()

                idx_cp(bslot, c, 0).wait()
                idx_cp(bslot, c, 1).wait()
                gat_cp(bslot, 0).start()
                gat_cp(bslot, 1).start()
                geo_compute(bslot)
                out_cp(bslot, c, 2).start()
                gat_cp(bslot, 0).wait()
                gat_cp(bslot, 1).wait()
                out_cp(bslot, c, 0).start()
                out_cp(bslot, c, 1).start()

                @pl.when(c + NBUF < RING)
                def _(_b=bslot, _c=c):
                    idx_cp(_b, _c + NBUF, 0).start()
                    idx_cp(_b, _c + NBUF, 1).start()

        for bslot in range(NBUF):
            out_cp(bslot, RING - NBUF + bslot, 0).wait()
            out_cp(bslot, RING - NBUF + bslot, 1).wait()
            out_cp(bslot, RING - NBUF + bslot, 2).wait()

    return k(a, b, px, py, pz, src, dst)


def _edge_body(ag_ref, bg_ref, geo_ref,
               w1dist_ref, w2_ref, b2_ref, cw1_ref, cb1_ref, cw2_ref,
               cb2_ref, msg_ref, dcw_ref):
    geo = geo_ref[...]
    d2 = geo[:, 3:4]
    pre = ag_ref[...] + bg_ref[...] + d2 * w1dist_ref[...]
    m = _silu(pre)
    msg = _silu(
        jnp.dot(m, w2_ref[...], preferred_element_type=jnp.float32,
                precision=lax.Precision.HIGHEST) + b2_ref[...])
    u = _silu(
        jnp.dot(msg, cw1_ref[...], preferred_element_type=jnp.float32,
                precision=lax.Precision.HIGHEST) + cb1_ref[...])
    cwv = jnp.dot(u, cw2_ref[...], preferred_element_type=jnp.float32,
                  precision=lax.Precision.HIGHEST) + cb2_ref[...]
    cw = jnp.clip(cwv[:, :1], -CLAMP, CLAMP)
    msg_ref[...] = msg
    lane = lax.broadcasted_iota(jnp.int32, geo.shape, 1)
    dcw_ref[...] = jnp.where(lane < 3, geo * cw, 0.0)


def _edge_mlp(ag, bg, geo, w1dist, w2, b2, cw1, cb1, cw2p, cb2p):
    blk = 2000
    grid = (E // blk,)
    wspec = lambda shape: pl.BlockSpec(shape, lambda i: (0, 0))
    return pl.pallas_call(
        _edge_body,
        grid=grid,
        in_specs=[
            pl.BlockSpec((blk, D), lambda i: (i, 0)),
            pl.BlockSpec((blk, D), lambda i: (i, 0)),
            pl.BlockSpec((blk, P16), lambda i: (i, 0)),
            wspec((1, D)),
            wspec((D, D)),
            wspec((1, D)),
            wspec((D, D)),
            wspec((1, D)),
            wspec((D, 8)),
            wspec((1, 8)),
        ],
        out_specs=[
            pl.BlockSpec((blk, D), lambda i: (i, 0)),
            pl.BlockSpec((blk, P16), lambda i: (i, 0)),
        ],
        out_shape=[
            jax.ShapeDtypeStruct((E, D), jnp.float32),
            jax.ShapeDtypeStruct((E, P16), jnp.float32),
        ],
    )(ag, bg, geo, w1dist, w2, b2, cw1, cb1, cw2p, cb2p)


def _sc_scatter(msg, dcw, dst, z128):
    mesh = plsc.VectorSubcoreMesh(core_axis_name="c", subcore_axis_name="s")
    L = 16

    @functools.partial(
        pl.kernel,
        mesh=mesh,
        compiler_params=_sc_cp(),
        out_type=[
            jax.ShapeDtypeStruct((NC, N, D), jnp.float32),
            jax.ShapeDtypeStruct((NC, N, D), jnp.float32),
        ],
        scratch_types=[
            pltpu.VMEM_SHARED((N, D), jnp.float32),
            pltpu.VMEM((CHUNK,), jnp.int32),
            pltpu.VMEM((CHUNK, D), jnp.float32),
            pltpu.VMEM((CHUNK, P16), jnp.float32),
            pltpu.VMEM((CHUNK, D), jnp.float32),
        ],
    )
    def k(msg_hbm, dcw_hbm, dst_hbm, z128_hbm,
          outa_hbm, outb_hbm, acc, didx_v, m_v, d_v, d128_v):
        cid = lax.axis_index("c")
        sid = lax.axis_index("s")
        base = cid * EPC + sid * EPT
        rows = pl.ds(sid * NPTA, NPTA)
        tail = pl.ds(NS * NPTA, N - NS * NPTA)
        zeros16 = jnp.zeros((L,), jnp.float32)

        @pl.loop(0, CHUNK)
        def _(i):
            for j in range(D // L):
                d128_v[i, pl.ds(j * L, L)] = zeros16

        def zero_acc():
            pltpu.sync_copy(z128_hbm.at[rows], acc.at[rows])

            @pl.when(sid == NS - 1)
            def _():
                pltpu.sync_copy(z128_hbm.at[tail], acc.at[tail])

        def drain(out_hbm):
            pltpu.sync_copy(acc.at[rows], out_hbm.at[cid].at[rows])

            @pl.when(sid == NS - 1)
            def _():
                pltpu.sync_copy(acc.at[tail], out_hbm.at[cid].at[tail])

        zero_acc()
        plsc.subcore_barrier()

        @pl.loop(0, NCHUNK)
        def _(c):
            off = base + c * CHUNK
            pltpu.sync_copy(dst_hbm.at[pl.ds(off, CHUNK)], didx_v)
            pltpu.sync_copy(msg_hbm.at[pl.ds(off, CHUNK)], m_v)
            pltpu.sync_copy(m_v, acc.at[didx_v], add=True)

        plsc.subcore_barrier()
        drain(outa_hbm)

        zero_acc()
        plsc.subcore_barrier()

        @pl.loop(0, NCHUNK)
        def _(c):
            off = base + c * CHUNK
            pltpu.sync_copy(dst_hbm.at[pl.ds(off, CHUNK)], didx_v)
            pltpu.sync_copy(dcw_hbm.at[pl.ds(off, CHUNK)], d_v)

            @pl.loop(0, CHUNK)
            def _(i):
                d128_v[i, pl.ds(0, L)] = d_v[i, pl.ds(0, L)]

            pltpu.sync_copy(d128_v, acc.at[didx_v], add=True)

        plsc.subcore_barrier()
        drain(outb_hbm)

    return k(msg, dcw, dst, z128)


def _node_body(h_ref, p16_ref, pa0_ref, pa1_ref, pb0_ref, pb1_ref,
               nw1h_ref, nw1m_ref, nb1_ref, nw2_ref, nb2_ref,
               hn_ref, pn_ref):
    h = h_ref[...]
    magg = pa0_ref[...] + pa1_ref[...]
    t = _silu(
        jnp.dot(h, nw1h_ref[...], preferred_element_type=jnp.float32,
                precision=lax.Precision.HIGHEST)
        + jnp.dot(magg, nw1m_ref[...], preferred_element_type=jnp.float32,
                  precision=lax.Precision.HIGHEST)
        + nb1_ref[...])
    hn_ref[...] = h + jnp.dot(
        t, nw2_ref[...], preferred_element_type=jnp.float32,
        precision=lax.Precision.HIGHEST) + nb2_ref[...]
    pn_ref[...] = p16_ref[...] + pb0_ref[:, :P16] + pb1_ref[:, :P16]


def _node_mlp(h, pos16, pa0, pa1, pb0, pb1, nw1h, nw1m, nb1, nw2, nb2):
    blk = 1000
    grid = (N // blk,)
    wspec = lambda shape: pl.BlockSpec(shape, lambda i: (0, 0))
    return pl.pallas_call(
        _node_body,
        grid=grid,
        in_specs=[
            pl.BlockSpec((blk, D), lambda i: (i, 0)),
            pl.BlockSpec((blk, P16), lambda i: (i, 0)),
            pl.BlockSpec((blk, D), lambda i: (i, 0)),
            pl.BlockSpec((blk, D), lambda i: (i, 0)),
            pl.BlockSpec((blk, D), lambda i: (i, 0)),
            pl.BlockSpec((blk, D), lambda i: (i, 0)),
            wspec((D, D)),
            wspec((D, D)),
            wspec((1, D)),
            wspec((D, D)),
            wspec((1, D)),
        ],
        out_specs=[
            pl.BlockSpec((blk, D), lambda i: (i, 0)),
            pl.BlockSpec((blk, P16), lambda i: (i, 0)),
        ],
        out_shape=[
            jax.ShapeDtypeStruct((N, D), jnp.float32),
            jax.ShapeDtypeStruct((N, P16), jnp.float32),
        ],
    )(h, pos16, pa0, pa1, pb0, pb1, nw1h, nw1m, nb1, nw2, nb2)


def kernel(h, pos, edge_index, msg_W1, msg_b1, msg_W2, msg_b2,
           coord_W1, coord_b1, coord_W2, coord_b2,
           node_W1, node_b1, node_W2, node_b2):
    src = edge_index[0]
    dst = edge_index[1]

    w1s = msg_W1[:D]
    w1d = msg_W1[D:2 * D]
    w1dist = msg_W1[2 * D].reshape(1, D)
    b1 = msg_b1.reshape(1, D)
    b2 = msg_b2.reshape(1, D)
    cb1 = coord_b1.reshape(1, D)
    cw2p = jnp.pad(coord_W2, ((0, 0), (0, 7)))
    cb2p = jnp.pad(coord_b2.reshape(1, 1), ((0, 0), (0, 7)))
    nw1h = node_W1[:D]
    nw1m = node_W1[D:]
    nb1 = node_b1.reshape(1, D)
    nb2 = node_b2.reshape(1, D)

    pos16 = jnp.pad(pos, ((0, 0), (0, P16 - 3)))
    z128 = jnp.zeros((N, D), jnp.float32)

    a, b = _preproj(h, w1s, w1d, b1)
    px = jnp.asarray(pos[:, 0])
    py = jnp.asarray(pos[:, 1])
    pz = jnp.asarray(pos[:, 2])
    ag, bg, geo = _sc_gather(a, b, px, py, pz, src, dst)
    msg, dcw = _edge_mlp(ag, bg, geo, w1dist, msg_W2, b2,
                         coord_W1, cb1, cw2p, cb2p)
    parts_a, parts_b = _sc_scatter(msg, dcw, dst, z128)
    h_new, pos16_new = _node_mlp(h, pos16, parts_a[0], parts_a[1],
                                 parts_b[0], parts_b[1],
                                 nw1h, nw1m, nb1, node_W2, nb2)
    return (h_new, pos16_new[:, :3])


# speedup vs baseline: 3.4020x; 1.1422x over previous
import dataclasses
import functools

import jax
import jax.numpy as jnp
from jax import lax
from jax.experimental import pallas as pl
from jax.experimental.pallas import tpu as pltpu
from jax.experimental.pallas import tpu_sc as plsc

N = 10000
E = 320000
D = 128
P16 = 16
CLAMP = 10.0

NC = 2
NS = 16
NW = NC * NS
EPT = E // NW
CHUNK = 80
NCHUNK = EPT // CHUNK

EPC = E // NC
NPTA = 624


def _silu(x):
    return x * jax.nn.sigmoid(x)


def _preproj_body(h_ref, w1s_ref, w1d_ref, b1_ref, a_ref, b_ref):
    h = h_ref[...]
    a_ref[...] = (
        jnp.dot(h, w1s_ref[...], preferred_element_type=jnp.float32,
                precision=lax.Precision.HIGHEST)
        + b1_ref[...]
    )
    b_ref[...] = jnp.dot(h, w1d_ref[...], preferred_element_type=jnp.float32,
                         precision=lax.Precision.HIGHEST)


def _preproj(h, w1s, w1d, b1):
    blk = 1000
    grid = (N // blk,)
    return pl.pallas_call(
        _preproj_body,
        grid=grid,
        in_specs=[
            pl.BlockSpec((blk, D), lambda i: (i, 0)),
            pl.BlockSpec((D, D), lambda i: (0, 0)),
            pl.BlockSpec((D, D), lambda i: (0, 0)),
            pl.BlockSpec((1, D), lambda i: (0, 0)),
        ],
        out_specs=[
            pl.BlockSpec((blk, D), lambda i: (i, 0)),
            pl.BlockSpec((blk, D), lambda i: (i, 0)),
        ],
        out_shape=[
            jax.ShapeDtypeStruct((N, D), jnp.float32),
            jax.ShapeDtypeStruct((N, D), jnp.float32),
        ],
    )(h, w1s, w1d, b1)


def _sc_cp():
    cp = pltpu.CompilerParams()
    if "needs_layout_passes" in pltpu.CompilerParams.__dataclass_fields__:
        cp = dataclasses.replace(cp, needs_layout_passes=False)
    return cp


NBUF = 3
RING = (NCHUNK // NBUF) * NBUF


def _sc_gather(a, b, px, py, pz, src, dst):
    mesh = plsc.VectorSubcoreMesh(core_axis_name="c", subcore_axis_name="s")
    L = 16
    NG = CHUNK // L

    scratch = (
        [pltpu.VMEM((CHUNK,), jnp.int32) for _ in range(2 * NBUF)]
        + [pltpu.VMEM((CHUNK, D), jnp.float32) for _ in range(2 * NBUF)]
        + [pltpu.VMEM((CHUNK, P16), jnp.float32) for _ in range(NBUF)]
        + [pltpu.VMEM((N,), jnp.float32) for _ in range(3)]
        + [pltpu.SemaphoreType.DMA for _ in range(3 * NBUF)]
    )

    @functools.partial(
        pl.kernel,
        mesh=mesh,
        compiler_params=_sc_cp(),
        out_type=[
            jax.ShapeDtypeStruct((E, D), jnp.float32),
            jax.ShapeDtypeStruct((E, D), jnp.float32),
            jax.ShapeDtypeStruct((E, P16), jnp.float32),
        ],
        scratch_types=scratch,
    )
    def k(a_hbm, b_hbm, px_hbm, py_hbm, pz_hbm, src_hbm, dst_hbm,
          ag_hbm, bg_hbm, geo_hbm, *sc):
        sidx = sc[0:NBUF]
        didx = sc[NBUF:2 * NBUF]
        av = sc[2 * NBUF:3 * NBUF]
        bv = sc[3 * NBUF:4 * NBUF]
        geov = sc[4 * NBUF:5 * NBUF]
        px_v, py_v, pz_v = sc[5 * NBUF:5 * NBUF + 3]
        sem_i = sc[5 * NBUF + 3:5 * NBUF + 3 + NBUF]
        sem_g = sc[5 * NBUF + 3 + NBUF:5 * NBUF + 3 + 2 * NBUF]
        sem_o = sc[5 * NBUF + 3 + 2 * NBUF:5 * NBUF + 3 + 3 * NBUF]

        wid = lax.axis_index("s") * NC + lax.axis_index("c")
        base = wid * EPT
        zeros16 = jnp.zeros((L,), jnp.float32)

        pltpu.sync_copy(px_hbm, px_v)
        pltpu.sync_copy(py_hbm, py_v)
        pltpu.sync_copy(pz_hbm, pz_v)

        for bslot in range(NBUF):
            @pl.loop(0, CHUNK)
            def _(i, _b=bslot):
                geov[_b][i, :] = zeros16

        def idx_cp(bslot, c, which):
            off = base + c * CHUNK
            hbm = src_hbm if which == 0 else dst_hbm
            buf = sidx[bslot] if which == 0 else didx[bslot]
            return pltpu.make_async_copy(hbm.at[pl.ds(off, CHUNK)], buf,
                                         sem_i[bslot])

        def gat_cp(bslot, which):
            if which == 0:
                return pltpu.make_async_copy(a_hbm.at[sidx[bslot]], av[bslot],
                                             sem_g[bslot])
            return pltpu.make_async_copy(b_hbm.at[didx[bslot]], bv[bslot],
                                         sem_g[bslot])

        def out_cp(bslot, c, which):
            off = base + c * CHUNK
            src_v, dst_h = [(av[bslot], ag_hbm), (bv[bslot], bg_hbm),
                            (geov[bslot], geo_hbm)][which]
            return pltpu.make_async_copy(src_v, dst_h.at[pl.ds(off, CHUNK)],
                                         sem_o[bslot])

        def geo_compute(bslot):
            for g in range(NG):
                si = sidx[bslot][pl.ds(g * L, L)]
                di = didx[bslot][pl.ds(g * L, L)]
                dx = plsc.load_gather(px_v, [si]) - plsc.load_gather(px_v, [di])
                dy = plsc.load_gather(py_v, [si]) - plsc.load_gather(py_v, [di])
                dz = plsc.load_gather(pz_v, [si]) - plsc.load_gather(pz_v, [di])
                d2 = dx * dx + dy * dy + dz * dz
                rows = g * L + lax.iota(jnp.int32, L)
                plsc.store_scatter(geov[bslot],
                                   [rows, jnp.full((L,), 0, jnp.int32)], dx)
                plsc.store_scatter(geov[bslot],
                                   [rows, jnp.full((L,), 1, jnp.int32)], dy)
                plsc.store_scatter(geov[bslot],
                                   [rows, jnp.full((L,), 2, jnp.int32)], dz)
                plsc.store_scatter(geov[bslot],
                                   [rows, jnp.full((L,), 3, jnp.int32)], d2)

        for c in range(RING, NCHUNK):
            idx_cp(0, c, 0).start()
            idx_cp(0, c, 1).start()
            idx_cp(0, c, 0).wait()
            idx_cp(0, c, 1).wait()
            gat_cp(0, 0).start()
            gat_cp(0, 1).start()
            geo_compute(0)
            gat_cp(0, 0).wait()
            gat_cp(0, 1).wait()
            out_cp(0, c, 0).start()
            out_cp(0, c, 1).start()
            out_cp(0, c, 2).start()
            out_cp(0, c, 0).wait()
            out_cp(0, c, 1).wait()
            out_cp(0, c, 2).wait()

        for bslot in range(NBUF):
            idx_cp(bslot, bslot, 0).start()
            idx_cp(bslot, bslot, 1).start()

        @pl.loop(0, RING // NBUF)
        def _(s):
            for bslot in range(NBUF):
                c = s * NBUF + bslot

                @pl.when(s > 0)
                def _(_b=bslot, _c=c):
                    out_cp(_b, _c - NBUF, 0).wait()
                    out_cp(_b, _c - NBUF, 1).wait()
                    out_cp(_b, _c - NBUF, 2).wait()

                idx_cp(bslot, c, 0).wait()
                idx_cp(bslot, c, 1).wait()
                gat_cp(bslot, 0).start()
                gat_cp(bslot, 1).start()
                geo_compute(bslot)
                out_cp(bslot, c, 2).start()
                gat_cp(bslot, 0).wait()
                gat_cp(bslot, 1).wait()
                out_cp(bslot, c, 0).start()
                out_cp(bslot, c, 1).start()

                @pl.when(c + NBUF < RING)
                def _(_b=bslot, _c=c):
                    idx_cp(_b, _c + NBUF, 0).start()
                    idx_cp(_b, _c + NBUF, 1).start()

        for bslot in range(NBUF):
            out_cp(bslot, RING - NBUF + bslot, 0).wait()
            out_cp(bslot, RING - NBUF + bslot, 1).wait()
            out_cp(bslot, RING - NBUF + bslot, 2).wait()

    return k(a, b, px, py, pz, src, dst)


def _edge_body(ag_ref, bg_ref, geo_ref,
               w1dist_ref, w2_ref, b2_ref, cw1_ref, cb1_ref, cw2_ref,
               cb2_ref, msg_ref, dcw_ref):
    geo = geo_ref[...]
    d2 = geo[:, 3:4]
    pre = ag_ref[...] + bg_ref[...] + d2 * w1dist_ref[...]
    m = _silu(pre)
    msg = _silu(
        jnp.dot(m, w2_ref[...], preferred_element_type=jnp.float32,
                precision=lax.Precision.HIGHEST) + b2_ref[...])
    u = _silu(
        jnp.dot(msg, cw1_ref[...], preferred_element_type=jnp.float32,
                precision=lax.Precision.HIGHEST) + cb1_ref[...])
    cwv = jnp.dot(u, cw2_ref[...], preferred_element_type=jnp.float32,
                  precision=lax.Precision.HIGHEST) + cb2_ref[...]
    cw = jnp.clip(cwv[:, :1], -CLAMP, CLAMP)
    msg_ref[...] = msg
    lane = lax.broadcasted_iota(jnp.int32, geo.shape, 1)
    dcw_ref[...] = jnp.where(lane < 3, geo * cw, 0.0)


def _edge_mlp(ag, bg, geo, w1dist, w2, b2, cw1, cb1, cw2p, cb2p):
    blk = 2000
    grid = (E // blk,)
    wspec = lambda shape: pl.BlockSpec(shape, lambda i: (0, 0))
    return pl.pallas_call(
        _edge_body,
        grid=grid,
        in_specs=[
            pl.BlockSpec((blk, D), lambda i: (i, 0)),
            pl.BlockSpec((blk, D), lambda i: (i, 0)),
            pl.BlockSpec((blk, P16), lambda i: (i, 0)),
            wspec((1, D)),
            wspec((D, D)),
            wspec((1, D)),
            wspec((D, D)),
            wspec((1, D)),
            wspec((D, 8)),
            wspec((1, 8)),
        ],
        out_specs=[
            pl.BlockSpec((blk, D), lambda i: (i, 0)),
            pl.BlockSpec((blk, P16), lambda i: (i, 0)),
        ],
        out_shape=[
            jax.ShapeDtypeStruct((E, D), jnp.float32),
            jax.ShapeDtypeStruct((E, P16), jnp.float32),
        ],
    )(ag, bg, geo, w1dist, w2, b2, cw1, cb1, cw2p, cb2p)


def _sc_scatter(msg, dcw, dst, z128):
    mesh = plsc.VectorSubcoreMesh(core_axis_name="c", subcore_axis_name="s")
    L = 16
    NBUF = 2
    RING = (NCHUNK // NBUF) * NBUF

    scratch = (
        [pltpu.VMEM_SHARED((N, D), jnp.float32)]
        + [pltpu.VMEM((CHUNK,), jnp.int32) for _ in range(NBUF)]
        + [pltpu.VMEM((CHUNK, D), jnp.float32) for _ in range(NBUF)]
        + [pltpu.VMEM((CHUNK, P16), jnp.float32) for _ in range(NBUF)]
        + [pltpu.SemaphoreType.DMA for _ in range(2 * NBUF)]
    )

    @functools.partial(
        pl.kernel,
        mesh=mesh,
        compiler_params=_sc_cp(),
        out_type=[
            jax.ShapeDtypeStruct((NC, N, D), jnp.float32),
            jax.ShapeDtypeStruct((NC, N, D), jnp.float32),
        ],
        scratch_types=scratch,
    )
    def k(msg_hbm, dcw_hbm, dst_hbm, z128_hbm, outa_hbm, outb_hbm, *sc):
        acc = sc[0]
        didx = sc[1:1 + NBUF]
        mv = sc[1 + NBUF:1 + 2 * NBUF]
        dv = sc[1 + 2 * NBUF:1 + 3 * NBUF]
        sem_i = sc[1 + 3 * NBUF:1 + 4 * NBUF]
        sem_a = sc[1 + 4 * NBUF:1 + 5 * NBUF]

        cid = lax.axis_index("c")
        sid = lax.axis_index("s")
        base = cid * EPC + sid * EPT
        rows = pl.ds(sid * NPTA, NPTA)
        tail = pl.ds(NS * NPTA, N - NS * NPTA)
        zeros16 = jnp.zeros((L,), jnp.float32)

        def zero_acc():
            pltpu.sync_copy(z128_hbm.at[rows], acc.at[rows])

            @pl.when(sid == NS - 1)
            def _():
                pltpu.sync_copy(z128_hbm.at[tail], acc.at[tail])

        def drain(out_hbm):
            pltpu.sync_copy(acc.at[rows], out_hbm.at[cid].at[rows])

            @pl.when(sid == NS - 1)
            def _():
                pltpu.sync_copy(acc.at[tail], out_hbm.at[cid].at[tail])

        def idx_cp(bslot, c):
            off = base + c * CHUNK
            return pltpu.make_async_copy(dst_hbm.at[pl.ds(off, CHUNK)],
                                         didx[bslot], sem_i[bslot])

        def pay_cp(bslot, c, phase):
            off = base + c * CHUNK
            if phase == 0:
                return pltpu.make_async_copy(msg_hbm.at[pl.ds(off, CHUNK)],
                                             mv[bslot], sem_i[bslot])
            return pltpu.make_async_copy(dcw_hbm.at[pl.ds(off, CHUNK)],
                                         dv[bslot], sem_i[bslot])

        def add_start(bslot):
            pltpu.async_copy(mv[bslot], acc.at[didx[bslot]],
                             sem_a[bslot], add=True)

        def add_wait(bslot):
            pltpu.make_async_copy(mv[bslot], acc.at[didx[bslot]],
                                  sem_a[bslot]).wait()

        def expand(bslot):
            @pl.loop(0, CHUNK)
            def _(i):
                mv[bslot][i, pl.ds(0, L)] = dv[bslot][i, pl.ds(0, L)]

        def phase(ph):
            zero_acc()
            if ph == 1:
                for bslot in range(NBUF):
                    @pl.loop(0, CHUNK)
                    def _(i, _b=bslot):
                        for j in range(1, D // L):
                            mv[_b][i, pl.ds(j * L, L)] = zeros16
            plsc.subcore_barrier()

            for c in range(RING, NCHUNK):
                idx_cp(0, c).start()
                pay_cp(0, c, ph).start()
                idx_cp(0, c).wait()
                pay_cp(0, c, ph).wait()
                if ph == 1:
                    expand(0)
                add_start(0)
                add_wait(0)

            idx_cp(0, 0).start()
            pay_cp(0, 0, ph).start()

            @pl.loop(0, RING // 2)
            def _(s):
                c0 = 2 * s
                idx_cp(0, c0).wait()
                pay_cp(0, c0, ph).wait()
                if ph == 1:
                    expand(0)
                add_start(0)

                @pl.when(s > 0)
                def _():
                    add_wait(1)

                idx_cp(1, c0 + 1).start()
                pay_cp(1, c0 + 1, ph).start()

                idx_cp(1, c0 + 1).wait()
                pay_cp(1, c0 + 1, ph).wait()
                if ph == 1:
                    expand(1)
                add_start(1)
                add_wait(0)

                @pl.when(s < RING // 2 - 1)
                def _(_c=c0 + 2):
                    idx_cp(0, _c).start()
                    pay_cp(0, _c, ph).start()

            add_wait(1)
            plsc.subcore_barrier()

        phase(0)
        drain(outa_hbm)
        phase(1)
        drain(outb_hbm)

    return k(msg, dcw, dst, z128)


def _node_body(h_ref, p16_ref, pa0_ref, pa1_ref, pb0_ref, pb1_ref,
               nw1h_ref, nw1m_ref, nb1_ref, nw2_ref, nb2_ref,
               hn_ref, pn_ref):
    h = h_ref[...]
    magg = pa0_ref[...] + pa1_ref[...]
    t = _silu(
        jnp.dot(h, nw1h_ref[...], preferred_element_type=jnp.float32,
                precision=lax.Precision.HIGHEST)
        + jnp.dot(magg, nw1m_ref[...], preferred_element_type=jnp.float32,
                  precision=lax.Precision.HIGHEST)
        + nb1_ref[...])
    hn_ref[...] = h + jnp.dot(
        t, nw2_ref[...], preferred_element_type=jnp.float32,
        precision=lax.Precision.HIGHEST) + nb2_ref[...]
    pn_ref[...] = p16_ref[...] + pb0_ref[:, :P16] + pb1_ref[:, :P16]


def _node_mlp(h, pos16, pa0, pa1, pb0, pb1, nw1h, nw1m, nb1, nw2, nb2):
    blk = 1000
    grid = (N // blk,)
    wspec = lambda shape: pl.BlockSpec(shape, lambda i: (0, 0))
    return pl.pallas_call(
        _node_body,
        grid=grid,
        in_specs=[
            pl.BlockSpec((blk, D), lambda i: (i, 0)),
            pl.BlockSpec((blk, P16), lambda i: (i, 0)),
            pl.BlockSpec((blk, D), lambda i: (i, 0)),
            pl.BlockSpec((blk, D), lambda i: (i, 0)),
            pl.BlockSpec((blk, D), lambda i: (i, 0)),
            pl.BlockSpec((blk, D), lambda i: (i, 0)),
            wspec((D, D)),
            wspec((D, D)),
            wspec((1, D)),
            wspec((D, D)),
            wspec((1, D)),
        ],
        out_specs=[
            pl.BlockSpec((blk, D), lambda i: (i, 0)),
            pl.BlockSpec((blk, P16), lambda i: (i, 0)),
        ],
        out_shape=[
            jax.ShapeDtypeStruct((N, D), jnp.float32),
            jax.ShapeDtypeStruct((N, P16), jnp.float32),
        ],
    )(h, pos16, pa0, pa1, pb0, pb1, nw1h, nw1m, nb1, nw2, nb2)


def kernel(h, pos, edge_index, msg_W1, msg_b1, msg_W2, msg_b2,
           coord_W1, coord_b1, coord_W2, coord_b2,
           node_W1, node_b1, node_W2, node_b2):
    src = edge_index[0]
    dst = edge_index[1]

    w1s = msg_W1[:D]
    w1d = msg_W1[D:2 * D]
    w1dist = msg_W1[2 * D].reshape(1, D)
    b1 = msg_b1.reshape(1, D)
    b2 = msg_b2.reshape(1, D)
    cb1 = coord_b1.reshape(1, D)
    cw2p = jnp.pad(coord_W2, ((0, 0), (0, 7)))
    cb2p = jnp.pad(coord_b2.reshape(1, 1), ((0, 0), (0, 7)))
    nw1h = node_W1[:D]
    nw1m = node_W1[D:]
    nb1 = node_b1.reshape(1, D)
    nb2 = node_b2.reshape(1, D)

    pos16 = jnp.pad(pos, ((0, 0), (0, P16 - 3)))
    z128 = jnp.zeros((N, D), jnp.float32)

    a, b = _preproj(h, w1s, w1d, b1)
    px = jnp.asarray(pos[:, 0])
    py = jnp.asarray(pos[:, 1])
    pz = jnp.asarray(pos[:, 2])
    ag, bg, geo = _sc_gather(a, b, px, py, pz, src, dst)
    msg, dcw = _edge_mlp(ag, bg, geo, w1dist, msg_W2, b2,
                         coord_W1, cb1, cw2p, cb2p)
    parts_a, parts_b = _sc_scatter(msg, dcw, dst, z128)
    h_new, pos16_new = _node_mlp(h, pos16, parts_a[0], parts_a[1],
                                 parts_b[0], parts_b[1],
                                 nw1h, nw1m, nb1, node_W2, nb2)
    return (h_new, pos16_new[:, :3])


# speedup vs baseline: 4.7369x; 1.3924x over previous
import dataclasses
import functools

import jax
import jax.numpy as jnp
from jax import lax
from jax.experimental import pallas as pl
from jax.experimental.pallas import tpu as pltpu
from jax.experimental.pallas import tpu_sc as plsc

N = 10000
E = 320000
D = 128
P16 = 16
CLAMP = 10.0

NC = 2
NS = 16
NW = NC * NS
EPT = E // NW
CHUNK = 80
NCHUNK = EPT // CHUNK

EPC = E // NC
NPTA = 624


def _silu(x):
    return x * jax.nn.sigmoid(x)


def _preproj_body(h_ref, w1s_ref, w1d_ref, b1_ref, a_ref, b_ref):
    h = h_ref[...]
    a_ref[...] = (
        jnp.dot(h, w1s_ref[...], preferred_element_type=jnp.float32,
                precision=lax.Precision.HIGHEST)
        + b1_ref[...]
    )
    b_ref[...] = jnp.dot(h, w1d_ref[...], preferred_element_type=jnp.float32,
                         precision=lax.Precision.HIGHEST)


def _preproj(h, w1s, w1d, b1):
    blk = 1000
    grid = (N // blk,)
    return pl.pallas_call(
        _preproj_body,
        grid=grid,
        in_specs=[
            pl.BlockSpec((blk, D), lambda i: (i, 0)),
            pl.BlockSpec((D, D), lambda i: (0, 0)),
            pl.BlockSpec((D, D), lambda i: (0, 0)),
            pl.BlockSpec((1, D), lambda i: (0, 0)),
        ],
        out_specs=[
            pl.BlockSpec((blk, D), lambda i: (i, 0)),
            pl.BlockSpec((blk, D), lambda i: (i, 0)),
        ],
        out_shape=[
            jax.ShapeDtypeStruct((N, D), jnp.float32),
            jax.ShapeDtypeStruct((N, D), jnp.float32),
        ],
    )(h, w1s, w1d, b1)


def _sc_cp():
    cp = pltpu.CompilerParams()
    if "needs_layout_passes" in pltpu.CompilerParams.__dataclass_fields__:
        cp = dataclasses.replace(cp, needs_layout_passes=False)
    return cp


NBUF = 3
RING = (NCHUNK // NBUF) * NBUF


def _sc_gather(a, b, px, py, pz, src, dst):
    mesh = plsc.VectorSubcoreMesh(core_axis_name="c", subcore_axis_name="s")
    L = 16
    NG = CHUNK // L

    scratch = (
        [pltpu.VMEM((CHUNK,), jnp.int32) for _ in range(2 * NBUF)]
        + [pltpu.VMEM((CHUNK, D), jnp.float32) for _ in range(2 * NBUF)]
        + [pltpu.VMEM((CHUNK, P16), jnp.float32) for _ in range(NBUF)]
        + [pltpu.VMEM((N,), jnp.float32) for _ in range(3)]
        + [pltpu.SemaphoreType.DMA for _ in range(3 * NBUF)]
    )

    @functools.partial(
        pl.kernel,
        mesh=mesh,
        compiler_params=_sc_cp(),
        out_type=[
            jax.ShapeDtypeStruct((E, D), jnp.float32),
            jax.ShapeDtypeStruct((E, D), jnp.float32),
            jax.ShapeDtypeStruct((E, P16), jnp.float32),
        ],
        scratch_types=scratch,
    )
    def k(a_hbm, b_hbm, px_hbm, py_hbm, pz_hbm, src_hbm, dst_hbm,
          ag_hbm, bg_hbm, geo_hbm, *sc):
        sidx = sc[0:NBUF]
        didx = sc[NBUF:2 * NBUF]
        av = sc[2 * NBUF:3 * NBUF]
        bv = sc[3 * NBUF:4 * NBUF]
        geov = sc[4 * NBUF:5 * NBUF]
        px_v, py_v, pz_v = sc[5 * NBUF:5 * NBUF + 3]
        sem_i = sc[5 * NBUF + 3:5 * NBUF + 3 + NBUF]
        sem_g = sc[5 * NBUF + 3 + NBUF:5 * NBUF + 3 + 2 * NBUF]
        sem_o = sc[5 * NBUF + 3 + 2 * NBUF:5 * NBUF + 3 + 3 * NBUF]

        wid = lax.axis_index("s") * NC + lax.axis_index("c")
        base = wid * EPT
        zeros16 = jnp.zeros((L,), jnp.float32)

        pltpu.sync_copy(px_hbm, px_v)
        pltpu.sync_copy(py_hbm, py_v)
        pltpu.sync_copy(pz_hbm, pz_v)

        for bslot in range(NBUF):
            @pl.loop(0, CHUNK)
            def _(i, _b=bslot):
                geov[_b][i, :] = zeros16

        def idx_cp(bslot, c, which):
            off = base + c * CHUNK
            hbm = src_hbm if which == 0 else dst_hbm
            buf = sidx[bslot] if which == 0 else didx[bslot]
            return pltpu.make_async_copy(hbm.at[pl.ds(off, CHUNK)], buf,
                                         sem_i[bslot])

        def gat_cp(bslot, which):
            if which == 0:
                return pltpu.make_async_copy(a_hbm.at[sidx[bslot]], av[bslot],
                                             sem_g[bslot])
            return pltpu.make_async_copy(b_hbm.at[didx[bslot]], bv[bslot],
                                         sem_g[bslot])

        def out_cp(bslot, c, which):
            off = base + c * CHUNK
            src_v, dst_h = [(av[bslot], ag_hbm), (bv[bslot], bg_hbm),
                            (geov[bslot], geo_hbm)][which]
            return pltpu.make_async_copy(src_v, dst_h.at[pl.ds(off, CHUNK)],
                                         sem_o[bslot])

        def geo_compute(bslot):
            for g in range(NG):
                si = sidx[bslot][pl.ds(g * L, L)]
                di = didx[bslot][pl.ds(g * L, L)]
                dx = plsc.load_gather(px_v, [si]) - plsc.load_gather(px_v, [di])
                dy = plsc.load_gather(py_v, [si]) - plsc.load_gather(py_v, [di])
                dz = plsc.load_gather(pz_v, [si]) - plsc.load_gather(pz_v, [di])
                d2 = dx * dx + dy * dy + dz * dz
                rows = g * L + lax.iota(jnp.int32, L)
                plsc.store_scatter(geov[bslot],
                                   [rows, jnp.full((L,), 0, jnp.int32)], dx)
                plsc.store_scatter(geov[bslot],
                                   [rows, jnp.full((L,), 1, jnp.int32)], dy)
                plsc.store_scatter(geov[bslot],
                                   [rows, jnp.full((L,), 2, jnp.int32)], dz)
                plsc.store_scatter(geov[bslot],
                                   [rows, jnp.full((L,), 3, jnp.int32)], d2)

        for c in range(RING, NCHUNK):
            idx_cp(0, c, 0).start()
            idx_cp(0, c, 1).start()
            idx_cp(0, c, 0).wait()
            idx_cp(0, c, 1).wait()
            gat_cp(0, 0).start()
            gat_cp(0, 1).start()
            geo_compute(0)
            gat_cp(0, 0).wait()
            gat_cp(0, 1).wait()
            out_cp(0, c, 0).start()
            out_cp(0, c, 1).start()
            out_cp(0, c, 2).start()
            out_cp(0, c, 0).wait()
            out_cp(0, c, 1).wait()
            out_cp(0, c, 2).wait()

        for bslot in range(NBUF):
            idx_cp(bslot, bslot, 0).start()
            idx_cp(bslot, bslot, 1).start()

        @pl.loop(0, RING // NBUF)
        def _(s):
            for bslot in range(NBUF):
                c = s * NBUF + bslot

                @pl.when(s > 0)
                def _(_b=bslot, _c=c):
                    out_cp(_b, _c - NBUF, 0).wait()
                    out_cp(_b, _c - NBUF, 1).wait()
                    out_cp(_b, _c - NBUF, 2).wait()

                idx_cp(bslot, c, 0).wait()
                idx_cp(bslot, c, 1).wait()
                gat_cp(bslot, 0).start()
                gat_cp(bslot, 1).start()
                geo_compute(bslot)
                out_cp(bslot, c, 2).start()
                gat_cp(bslot, 0).wait()
                gat_cp(bslot, 1).wait()
                out_cp(bslot, c, 0).start()
                out_cp(bslot, c, 1).start()

                @pl.when(c + NBUF < RING)
                def _(_b=bslot, _c=c):
                    idx_cp(_b, _c + NBUF, 0).start()
                    idx_cp(_b, _c + NBUF, 1).start()

        for bslot in range(NBUF):
            out_cp(bslot, RING - NBUF + bslot, 0).wait()
            out_cp(bslot, RING - NBUF + bslot, 1).wait()
            out_cp(bslot, RING - NBUF + bslot, 2).wait()

    return k(a, b, px, py, pz, src, dst)


def _edge_body(ag_ref, bg_ref, geo_ref,
               w1dist_ref, w2_ref, b2_ref, cw1_ref, cb1_ref, cw2_ref,
               cb2_ref, msg_ref, dcw_ref):
    geo = geo_ref[...]
    d2 = geo[:, 3:4]
    pre = ag_ref[...] + bg_ref[...] + d2 * w1dist_ref[...]
    m = _silu(pre)
    msg = _silu(
        jnp.dot(m, w2_ref[...], preferred_element_type=jnp.float32,
                precision=lax.Precision.DEFAULT) + b2_ref[...])
    u = _silu(
        jnp.dot(msg, cw1_ref[...], preferred_element_type=jnp.float32,
                precision=lax.Precision.DEFAULT) + cb1_ref[...])
    cwv = jnp.dot(u, cw2_ref[...], preferred_element_type=jnp.float32,
                  precision=lax.Precision.DEFAULT) + cb2_ref[...]
    cw = jnp.clip(cwv[:, :1], -CLAMP, CLAMP)
    msg_ref[...] = msg
    lane = lax.broadcasted_iota(jnp.int32, geo.shape, 1)
    dcw_ref[...] = jnp.where(lane < 3, geo * cw, 0.0)


def _edge_mlp(ag, bg, geo, w1dist, w2, b2, cw1, cb1, cw2p, cb2p):
    blk = 2000
    grid = (E // blk,)
    wspec = lambda shape: pl.BlockSpec(shape, lambda i: (0, 0))
    return pl.pallas_call(
        _edge_body,
        grid=grid,
        in_specs=[
            pl.BlockSpec((blk, D), lambda i: (i, 0)),
            pl.BlockSpec((blk, D), lambda i: (i, 0)),
            pl.BlockSpec((blk, P16), lambda i: (i, 0)),
            wspec((1, D)),
            wspec((D, D)),
            wspec((1, D)),
            wspec((D, D)),
            wspec((1, D)),
            wspec((D, 8)),
            wspec((1, 8)),
        ],
        out_specs=[
            pl.BlockSpec((blk, D), lambda i: (i, 0)),
            pl.BlockSpec((blk, P16), lambda i: (i, 0)),
        ],
        out_shape=[
            jax.ShapeDtypeStruct((E, D), jnp.float32),
            jax.ShapeDtypeStruct((E, P16), jnp.float32),
        ],
    )(ag, bg, geo, w1dist, w2, b2, cw1, cb1, cw2p, cb2p)


def _sc_scatter(msg, dcw, dst, z128):
    mesh = plsc.VectorSubcoreMesh(core_axis_name="c", subcore_axis_name="s")
    L = 16
    NBUF = 2
    RING = (NCHUNK // NBUF) * NBUF

    scratch = (
        [pltpu.VMEM_SHARED((N, D), jnp.float32)]
        + [pltpu.VMEM((CHUNK,), jnp.int32) for _ in range(NBUF)]
        + [pltpu.VMEM((CHUNK, D), jnp.float32) for _ in range(NBUF)]
        + [pltpu.VMEM((CHUNK, P16), jnp.float32) for _ in range(NBUF)]
        + [pltpu.SemaphoreType.DMA for _ in range(2 * NBUF)]
    )

    @functools.partial(
        pl.kernel,
        mesh=mesh,
        compiler_params=_sc_cp(),
        out_type=[
            jax.ShapeDtypeStruct((NC, N, D), jnp.float32),
            jax.ShapeDtypeStruct((NC, N, D), jnp.float32),
        ],
        scratch_types=scratch,
    )
    def k(msg_hbm, dcw_hbm, dst_hbm, z128_hbm, outa_hbm, outb_hbm, *sc):
        acc = sc[0]
        didx = sc[1:1 + NBUF]
        mv = sc[1 + NBUF:1 + 2 * NBUF]
        dv = sc[1 + 2 * NBUF:1 + 3 * NBUF]
        sem_i = sc[1 + 3 * NBUF:1 + 4 * NBUF]
        sem_a = sc[1 + 4 * NBUF:1 + 5 * NBUF]

        cid = lax.axis_index("c")
        sid = lax.axis_index("s")
        base = cid * EPC + sid * EPT
        rows = pl.ds(sid * NPTA, NPTA)
        tail = pl.ds(NS * NPTA, N - NS * NPTA)
        zeros16 = jnp.zeros((L,), jnp.float32)

        def zero_acc():
            pltpu.sync_copy(z128_hbm.at[rows], acc.at[rows])

            @pl.when(sid == NS - 1)
            def _():
                pltpu.sync_copy(z128_hbm.at[tail], acc.at[tail])

        def drain(out_hbm):
            pltpu.sync_copy(acc.at[rows], out_hbm.at[cid].at[rows])

            @pl.when(sid == NS - 1)
            def _():
                pltpu.sync_copy(acc.at[tail], out_hbm.at[cid].at[tail])

        def idx_cp(bslot, c):
            off = base + c * CHUNK
            return pltpu.make_async_copy(dst_hbm.at[pl.ds(off, CHUNK)],
                                         didx[bslot], sem_i[bslot])

        def pay_cp(bslot, c, phase):
            off = base + c * CHUNK
            if phase == 0:
                return pltpu.make_async_copy(msg_hbm.at[pl.ds(off, CHUNK)],
                                             mv[bslot], sem_i[bslot])
            return pltpu.make_async_copy(dcw_hbm.at[pl.ds(off, CHUNK)],
                                         dv[bslot], sem_i[bslot])

        def add_start(bslot):
            pltpu.async_copy(mv[bslot], acc.at[didx[bslot]],
                             sem_a[bslot], add=True)

        def add_wait(bslot):
            pltpu.make_async_copy(mv[bslot], acc.at[didx[bslot]],
                                  sem_a[bslot]).wait()

        def expand(bslot):
            @pl.loop(0, CHUNK)
            def _(i):
                mv[bslot][i, pl.ds(0, L)] = dv[bslot][i, pl.ds(0, L)]

        def phase(ph):
            zero_acc()
            if ph == 1:
                for bslot in range(NBUF):
                    @pl.loop(0, CHUNK)
                    def _(i, _b=bslot):
                        for j in range(1, D // L):
                            mv[_b][i, pl.ds(j * L, L)] = zeros16
            plsc.subcore_barrier()

            for c in range(RING, NCHUNK):
                idx_cp(0, c).start()
                pay_cp(0, c, ph).start()
                idx_cp(0, c).wait()
                pay_cp(0, c, ph).wait()
                if ph == 1:
                    expand(0)
                add_start(0)
                add_wait(0)

            idx_cp(0, 0).start()
            pay_cp(0, 0, ph).start()

            @pl.loop(0, RING // 2)
            def _(s):
                c0 = 2 * s
                idx_cp(0, c0).wait()
                pay_cp(0, c0, ph).wait()
                if ph == 1:
                    expand(0)
                add_start(0)

                @pl.when(s > 0)
                def _():
                    add_wait(1)

                idx_cp(1, c0 + 1).start()
                pay_cp(1, c0 + 1, ph).start()

                idx_cp(1, c0 + 1).wait()
                pay_cp(1, c0 + 1, ph).wait()
                if ph == 1:
                    expand(1)
                add_start(1)
                add_wait(0)

                @pl.when(s < RING // 2 - 1)
                def _(_c=c0 + 2):
                    idx_cp(0, _c).start()
                    pay_cp(0, _c, ph).start()

            add_wait(1)
            plsc.subcore_barrier()

        phase(0)
        drain(outa_hbm)
        phase(1)
        drain(outb_hbm)

    return k(msg, dcw, dst, z128)


def _node_body(h_ref, p16_ref, pa0_ref, pa1_ref, pb0_ref, pb1_ref,
               nw1h_ref, nw1m_ref, nb1_ref, nw2_ref, nb2_ref,
               hn_ref, pn_ref):
    h = h_ref[...]
    magg = pa0_ref[...] + pa1_ref[...]
    t = _silu(
        jnp.dot(h, nw1h_ref[...], preferred_element_type=jnp.float32,
                precision=lax.Precision.HIGHEST)
        + jnp.dot(magg, nw1m_ref[...], preferred_element_type=jnp.float32,
                  precision=lax.Precision.HIGHEST)
        + nb1_ref[...])
    hn_ref[...] = h + jnp.dot(
        t, nw2_ref[...], preferred_element_type=jnp.float32,
        precision=lax.Precision.HIGHEST) + nb2_ref[...]
    pn_ref[...] = p16_ref[...] + pb0_ref[:, :P16] + pb1_ref[:, :P16]


def _node_mlp(h, pos16, pa0, pa1, pb0, pb1, nw1h, nw1m, nb1, nw2, nb2):
    blk = 1000
    grid = (N // blk,)
    wspec = lambda shape: pl.BlockSpec(shape, lambda i: (0, 0))
    return pl.pallas_call(
        _node_body,
        grid=grid,
        in_specs=[
            pl.BlockSpec((blk, D), lambda i: (i, 0)),
            pl.BlockSpec((blk, P16), lambda i: (i, 0)),
            pl.BlockSpec((blk, D), lambda i: (i, 0)),
            pl.BlockSpec((blk, D), lambda i: (i, 0)),
            pl.BlockSpec((blk, D), lambda i: (i, 0)),
            pl.BlockSpec((blk, D), lambda i: (i, 0)),
            wspec((D, D)),
            wspec((D, D)),
            wspec((1, D)),
            wspec((D, D)),
            wspec((1, D)),
        ],
        out_specs=[
            pl.BlockSpec((blk, D), lambda i: (i, 0)),
            pl.BlockSpec((blk, P16), lambda i: (i, 0)),
        ],
        out_shape=[
            jax.ShapeDtypeStruct((N, D), jnp.float32),
            jax.ShapeDtypeStruct((N, P16), jnp.float32),
        ],
    )(h, pos16, pa0, pa1, pb0, pb1, nw1h, nw1m, nb1, nw2, nb2)


def kernel(h, pos, edge_index, msg_W1, msg_b1, msg_W2, msg_b2,
           coord_W1, coord_b1, coord_W2, coord_b2,
           node_W1, node_b1, node_W2, node_b2):
    src = edge_index[0]
    dst = edge_index[1]

    w1s = msg_W1[:D]
    w1d = msg_W1[D:2 * D]
    w1dist = msg_W1[2 * D].reshape(1, D)
    b1 = msg_b1.reshape(1, D)
    b2 = msg_b2.reshape(1, D)
    cb1 = coord_b1.reshape(1, D)
    cw2p = jnp.pad(coord_W2, ((0, 0), (0, 7)))
    cb2p = jnp.pad(coord_b2.reshape(1, 1), ((0, 0), (0, 7)))
    nw1h = node_W1[:D]
    nw1m = node_W1[D:]
    nb1 = node_b1.reshape(1, D)
    nb2 = node_b2.reshape(1, D)

    pos16 = jnp.pad(pos, ((0, 0), (0, P16 - 3)))
    z128 = jnp.zeros((N, D), jnp.float32)

    a, b = _preproj(h, w1s, w1d, b1)
    px = jnp.asarray(pos[:, 0])
    py = jnp.asarray(pos[:, 1])
    pz = jnp.asarray(pos[:, 2])
    ag, bg, geo = _sc_gather(a, b, px, py, pz, src, dst)
    msg, dcw = _edge_mlp(ag, bg, geo, w1dist, msg_W2, b2,
                         coord_W1, cb1, cw2p, cb2p)
    parts_a, parts_b = _sc_scatter(msg, dcw, dst, z128)
    h_new, pos16_new = _node_mlp(h, pos16, parts_a[0], parts_a[1],
                                 parts_b[0], parts_b[1],
                                 nw1h, nw1m, nb1, node_W2, nb2)
    return (h_new, pos16_new[:, :3])


# speedup vs baseline: 4.8296x; 1.0196x over previous
import dataclasses
import functools

import jax
import jax.numpy as jnp
from jax import lax
from jax.experimental import pallas as pl
from jax.experimental.pallas import tpu as pltpu
from jax.experimental.pallas import tpu_sc as plsc

N = 10000
E = 320000
D = 128
P16 = 16
CLAMP = 10.0

NC = 2
NS = 16
NW = NC * NS
EPT = E // NW
CHUNK = 80
NCHUNK = EPT // CHUNK

EPC = E // NC
NPTA = 624


def _silu(x):
    return x * jax.nn.sigmoid(x)


def _preproj_body(h_ref, w1s_ref, w1d_ref, b1_ref, a_ref, b_ref):
    h = h_ref[...]
    a_ref[...] = (
        jnp.dot(h, w1s_ref[...], preferred_element_type=jnp.float32,
                precision=lax.Precision.DEFAULT)
        + b1_ref[...]
    )
    b_ref[...] = jnp.dot(h, w1d_ref[...], preferred_element_type=jnp.float32,
                         precision=lax.Precision.DEFAULT)


def _preproj(h, w1s, w1d, b1):
    blk = 1000
    grid = (N // blk,)
    return pl.pallas_call(
        _preproj_body,
        grid=grid,
        in_specs=[
            pl.BlockSpec((blk, D), lambda i: (i, 0)),
            pl.BlockSpec((D, D), lambda i: (0, 0)),
            pl.BlockSpec((D, D), lambda i: (0, 0)),
            pl.BlockSpec((1, D), lambda i: (0, 0)),
        ],
        out_specs=[
            pl.BlockSpec((blk, D), lambda i: (i, 0)),
            pl.BlockSpec((blk, D), lambda i: (i, 0)),
        ],
        out_shape=[
            jax.ShapeDtypeStruct((N, D), jnp.float32),
            jax.ShapeDtypeStruct((N, D), jnp.float32),
        ],
    )(h, w1s, w1d, b1)


def _sc_cp():
    cp = pltpu.CompilerParams()
    if "needs_layout_passes" in pltpu.CompilerParams.__dataclass_fields__:
        cp = dataclasses.replace(cp, needs_layout_passes=False)
    return cp


NBUF = 3
RING = (NCHUNK // NBUF) * NBUF


def _sc_gather(a, b, px, py, pz, src, dst):
    mesh = plsc.VectorSubcoreMesh(core_axis_name="c", subcore_axis_name="s")
    L = 16
    NG = CHUNK // L

    scratch = (
        [pltpu.VMEM((CHUNK,), jnp.int32) for _ in range(2 * NBUF)]
        + [pltpu.VMEM((CHUNK, D), jnp.float32) for _ in range(2 * NBUF)]
        + [pltpu.VMEM((CHUNK, P16), jnp.float32) for _ in range(NBUF)]
        + [pltpu.VMEM((N,), jnp.float32) for _ in range(3)]
        + [pltpu.SemaphoreType.DMA for _ in range(3 * NBUF)]
    )

    @functools.partial(
        pl.kernel,
        mesh=mesh,
        compiler_params=_sc_cp(),
        out_type=[
            jax.ShapeDtypeStruct((E, D), jnp.float32),
            jax.ShapeDtypeStruct((E, D), jnp.float32),
            jax.ShapeDtypeStruct((E, P16), jnp.float32),
        ],
        scratch_types=scratch,
    )
    def k(a_hbm, b_hbm, px_hbm, py_hbm, pz_hbm, src_hbm, dst_hbm,
          ag_hbm, bg_hbm, geo_hbm, *sc):
        sidx = sc[0:NBUF]
        didx = sc[NBUF:2 * NBUF]
        av = sc[2 * NBUF:3 * NBUF]
        bv = sc[3 * NBUF:4 * NBUF]
        geov = sc[4 * NBUF:5 * NBUF]
        px_v, py_v, pz_v = sc[5 * NBUF:5 * NBUF + 3]
        sem_i = sc[5 * NBUF + 3:5 * NBUF + 3 + NBUF]
        sem_g = sc[5 * NBUF + 3 + NBUF:5 * NBUF + 3 + 2 * NBUF]
        sem_o = sc[5 * NBUF + 3 + 2 * NBUF:5 * NBUF + 3 + 3 * NBUF]

        wid = lax.axis_index("s") * NC + lax.axis_index("c")
        base = wid * EPT
        zeros16 = jnp.zeros((L,), jnp.float32)

        pltpu.sync_copy(px_hbm, px_v)
        pltpu.sync_copy(py_hbm, py_v)
        pltpu.sync_copy(pz_hbm, pz_v)

        for bslot in range(NBUF):
            @pl.loop(0, CHUNK)
            def _(i, _b=bslot):
                geov[_b][i, :] = zeros16

        def idx_cp(bslot, c, which):
            off = base + c * CHUNK
            hbm = src_hbm if which == 0 else dst_hbm
            buf = sidx[bslot] if which == 0 else didx[bslot]
            return pltpu.make_async_copy(hbm.at[pl.ds(off, CHUNK)], buf,
                                         sem_i[bslot])

        def gat_cp(bslot, which):
            if which == 0:
                return pltpu.make_async_copy(a_hbm.at[sidx[bslot]], av[bslot],
                                             sem_g[bslot])
            return pltpu.make_async_copy(b_hbm.at[didx[bslot]], bv[bslot],
                                         sem_g[bslot])

        def out_cp(bslot, c, which):
            off = base + c * CHUNK
            src_v, dst_h = [(av[bslot], ag_hbm), (bv[bslot], bg_hbm),
                            (geov[bslot], geo_hbm)][which]
            return pltpu.make_async_copy(src_v, dst_h.at[pl.ds(off, CHUNK)],
                                         sem_o[bslot])

        def geo_compute(bslot):
            for g in range(NG):
                si = sidx[bslot][pl.ds(g * L, L)]
                di = didx[bslot][pl.ds(g * L, L)]
                dx = plsc.load_gather(px_v, [si]) - plsc.load_gather(px_v, [di])
                dy = plsc.load_gather(py_v, [si]) - plsc.load_gather(py_v, [di])
                dz = plsc.load_gather(pz_v, [si]) - plsc.load_gather(pz_v, [di])
                d2 = dx * dx + dy * dy + dz * dz
                rows = g * L + lax.iota(jnp.int32, L)
                plsc.store_scatter(geov[bslot],
                                   [rows, jnp.full((L,), 0, jnp.int32)], dx)
                plsc.store_scatter(geov[bslot],
                                   [rows, jnp.full((L,), 1, jnp.int32)], dy)
                plsc.store_scatter(geov[bslot],
                                   [rows, jnp.full((L,), 2, jnp.int32)], dz)
                plsc.store_scatter(geov[bslot],
                                   [rows, jnp.full((L,), 3, jnp.int32)], d2)

        for c in range(RING, NCHUNK):
            idx_cp(0, c, 0).start()
            idx_cp(0, c, 1).start()
            idx_cp(0, c, 0).wait()
            idx_cp(0, c, 1).wait()
            gat_cp(0, 0).start()
            gat_cp(0, 1).start()
            geo_compute(0)
            gat_cp(0, 0).wait()
            gat_cp(0, 1).wait()
            out_cp(0, c, 0).start()
            out_cp(0, c, 1).start()
            out_cp(0, c, 2).start()
            out_cp(0, c, 0).wait()
            out_cp(0, c, 1).wait()
            out_cp(0, c, 2).wait()

        for bslot in range(NBUF):
            idx_cp(bslot, bslot, 0).start()
            idx_cp(bslot, bslot, 1).start()

        @pl.loop(0, RING // NBUF)
        def _(s):
            for bslot in range(NBUF):
                c = s * NBUF + bslot

                @pl.when(s > 0)
                def _(_b=bslot, _c=c):
                    out_cp(_b, _c - NBUF, 0).wait()
                    out_cp(_b, _c - NBUF, 1).wait()
                    out_cp(_b, _c - NBUF, 2).wait()

                idx_cp(bslot, c, 0).wait()
                idx_cp(bslot, c, 1).wait()
                gat_cp(bslot, 0).start()
                gat_cp(bslot, 1).start()
                geo_compute(bslot)
                out_cp(bslot, c, 2).start()
                gat_cp(bslot, 0).wait()
                gat_cp(bslot, 1).wait()
                out_cp(bslot, c, 0).start()
                out_cp(bslot, c, 1).start()

                @pl.when(c + NBUF < RING)
                def _(_b=bslot, _c=c):
                    idx_cp(_b, _c + NBUF, 0).start()
                    idx_cp(_b, _c + NBUF, 1).start()

        for bslot in range(NBUF):
            out_cp(bslot, RING - NBUF + bslot, 0).wait()
            out_cp(bslot, RING - NBUF + bslot, 1).wait()
            out_cp(bslot, RING - NBUF + bslot, 2).wait()

    return k(a, b, px, py, pz, src, dst)


def _edge_body(ag_ref, bg_ref, geo_ref,
               w1dist_ref, w2_ref, b2_ref, cw1_ref, cb1_ref, cw2_ref,
               cb2_ref, msg_ref, dcw_ref):
    geo = geo_ref[...]
    d2 = geo[:, 3:4]
    pre = ag_ref[...] + bg_ref[...] + d2 * w1dist_ref[...]
    m = _silu(pre)
    msg = _silu(
        jnp.dot(m, w2_ref[...], preferred_element_type=jnp.float32,
                precision=lax.Precision.DEFAULT) + b2_ref[...])
    u = _silu(
        jnp.dot(msg, cw1_ref[...], preferred_element_type=jnp.float32,
                precision=lax.Precision.DEFAULT) + cb1_ref[...])
    cwv = jnp.dot(u, cw2_ref[...], preferred_element_type=jnp.float32,
                  precision=lax.Precision.DEFAULT) + cb2_ref[...]
    cw = jnp.clip(cwv[:, :1], -CLAMP, CLAMP)
    msg_ref[...] = msg
    lane = lax.broadcasted_iota(jnp.int32, geo.shape, 1)
    dcw_ref[...] = jnp.where(lane < 3, geo * cw, 0.0)


def _edge_mlp(ag, bg, geo, w1dist, w2, b2, cw1, cb1, cw2p, cb2p):
    blk = 2000
    grid = (E // blk,)
    wspec = lambda shape: pl.BlockSpec(shape, lambda i: (0, 0))
    return pl.pallas_call(
        _edge_body,
        grid=grid,
        in_specs=[
            pl.BlockSpec((blk, D), lambda i: (i, 0)),
            pl.BlockSpec((blk, D), lambda i: (i, 0)),
            pl.BlockSpec((blk, P16), lambda i: (i, 0)),
            wspec((1, D)),
            wspec((D, D)),
            wspec((1, D)),
            wspec((D, D)),
            wspec((1, D)),
            wspec((D, 8)),
            wspec((1, 8)),
        ],
        out_specs=[
            pl.BlockSpec((blk, D), lambda i: (i, 0)),
            pl.BlockSpec((blk, P16), lambda i: (i, 0)),
        ],
        out_shape=[
            jax.ShapeDtypeStruct((E, D), jnp.float32),
            jax.ShapeDtypeStruct((E, P16), jnp.float32),
        ],
    )(ag, bg, geo, w1dist, w2, b2, cw1, cb1, cw2p, cb2p)


def _sc_scatter(msg, dcw, dst, z128):
    mesh = plsc.VectorSubcoreMesh(core_axis_name="c", subcore_axis_name="s")
    L = 16
    NBUF = 2
    RING = (NCHUNK // NBUF) * NBUF

    scratch = (
        [pltpu.VMEM_SHARED((N, D), jnp.float32)]
        + [pltpu.VMEM((CHUNK,), jnp.int32) for _ in range(NBUF)]
        + [pltpu.VMEM((CHUNK, D), jnp.float32) for _ in range(NBUF)]
        + [pltpu.VMEM((CHUNK, P16), jnp.float32) for _ in range(NBUF)]
        + [pltpu.SemaphoreType.DMA for _ in range(2 * NBUF)]
    )

    @functools.partial(
        pl.kernel,
        mesh=mesh,
        compiler_params=_sc_cp(),
        out_type=[
            jax.ShapeDtypeStruct((NC, N, D), jnp.float32),
            jax.ShapeDtypeStruct((NC, N, D), jnp.float32),
        ],
        scratch_types=scratch,
    )
    def k(msg_hbm, dcw_hbm, dst_hbm, z128_hbm, outa_hbm, outb_hbm, *sc):
        acc = sc[0]
        didx = sc[1:1 + NBUF]
        mv = sc[1 + NBUF:1 + 2 * NBUF]
        dv = sc[1 + 2 * NBUF:1 + 3 * NBUF]
        sem_i = sc[1 + 3 * NBUF:1 + 4 * NBUF]
        sem_a = sc[1 + 4 * NBUF:1 + 5 * NBUF]

        cid = lax.axis_index("c")
        sid = lax.axis_index("s")
        base = cid * EPC + sid * EPT
        rows = pl.ds(sid * NPTA, NPTA)
        tail = pl.ds(NS * NPTA, N - NS * NPTA)
        zeros16 = jnp.zeros((L,), jnp.float32)

        def zero_acc():
            pltpu.sync_copy(z128_hbm.at[rows], acc.at[rows])

            @pl.when(sid == NS - 1)
            def _():
                pltpu.sync_copy(z128_hbm.at[tail], acc.at[tail])

        def drain(out_hbm):
            pltpu.sync_copy(acc.at[rows], out_hbm.at[cid].at[rows])

            @pl.when(sid == NS - 1)
            def _():
                pltpu.sync_copy(acc.at[tail], out_hbm.at[cid].at[tail])

        def idx_cp(bslot, c):
            off = base + c * CHUNK
            return pltpu.make_async_copy(dst_hbm.at[pl.ds(off, CHUNK)],
                                         didx[bslot], sem_i[bslot])

        def pay_cp(bslot, c, phase):
            off = base + c * CHUNK
            if phase == 0:
                return pltpu.make_async_copy(msg_hbm.at[pl.ds(off, CHUNK)],
                                             mv[bslot], sem_i[bslot])
            return pltpu.make_async_copy(dcw_hbm.at[pl.ds(off, CHUNK)],
                                         dv[bslot], sem_i[bslot])

        def add_start(bslot):
            pltpu.async_copy(mv[bslot], acc.at[didx[bslot]],
                             sem_a[bslot], add=True)

        def add_wait(bslot):
            pltpu.make_async_copy(mv[bslot], acc.at[didx[bslot]],
                                  sem_a[bslot]).wait()

        def expand(bslot):
            @pl.loop(0, CHUNK)
            def _(i):
                mv[bslot][i, pl.ds(0, L)] = dv[bslot][i, pl.ds(0, L)]

        def phase(ph):
            zero_acc()
            if ph == 1:
                for bslot in range(NBUF):
                    @pl.loop(0, CHUNK)
                    def _(i, _b=bslot):
                        for j in range(1, D // L):
                            mv[_b][i, pl.ds(j * L, L)] = zeros16
            plsc.subcore_barrier()

            for c in range(RING, NCHUNK):
                idx_cp(0, c).start()
                pay_cp(0, c, ph).start()
                idx_cp(0, c).wait()
                pay_cp(0, c, ph).wait()
                if ph == 1:
                    expand(0)
                add_start(0)
                add_wait(0)

            idx_cp(0, 0).start()
            pay_cp(0, 0, ph).start()

            @pl.loop(0, RING // 2)
            def _(s):
                c0 = 2 * s
                idx_cp(0, c0).wait()
                pay_cp(0, c0, ph).wait()
                if ph == 1:
                    expand(0)
                add_start(0)

                @pl.when(s > 0)
                def _():
                    add_wait(1)

                idx_cp(1, c0 + 1).start()
                pay_cp(1, c0 + 1, ph).start()

                idx_cp(1, c0 + 1).wait()
                pay_cp(1, c0 + 1, ph).wait()
                if ph == 1:
                    expand(1)
                add_start(1)
                add_wait(0)

                @pl.when(s < RING // 2 - 1)
                def _(_c=c0 + 2):
                    idx_cp(0, _c).start()
                    pay_cp(0, _c, ph).start()

            add_wait(1)
            plsc.subcore_barrier()

        phase(0)
        drain(outa_hbm)
        phase(1)
        drain(outb_hbm)

    return k(msg, dcw, dst, z128)


def _node_body(h_ref, p16_ref, pa0_ref, pa1_ref, pb0_ref, pb1_ref,
               nw1h_ref, nw1m_ref, nb1_ref, nw2_ref, nb2_ref,
               hn_ref, pn_ref):
    h = h_ref[...]
    magg = pa0_ref[...] + pa1_ref[...]
    t = _silu(
        jnp.dot(h, nw1h_ref[...], preferred_element_type=jnp.float32,
                precision=lax.Precision.DEFAULT)
        + jnp.dot(magg, nw1m_ref[...], preferred_element_type=jnp.float32,
                  precision=lax.Precision.DEFAULT)
        + nb1_ref[...])
    hn_ref[...] = h + jnp.dot(
        t, nw2_ref[...], preferred_element_type=jnp.float32,
        precision=lax.Precision.DEFAULT) + nb2_ref[...]
    pn_ref[...] = p16_ref[...] + pb0_ref[:, :P16] + pb1_ref[:, :P16]


def _node_mlp(h, pos16, pa0, pa1, pb0, pb1, nw1h, nw1m, nb1, nw2, nb2):
    blk = 1000
    grid = (N // blk,)
    wspec = lambda shape: pl.BlockSpec(shape, lambda i: (0, 0))
    return pl.pallas_call(
        _node_body,
        grid=grid,
        in_specs=[
            pl.BlockSpec((blk, D), lambda i: (i, 0)),
            pl.BlockSpec((blk, P16), lambda i: (i, 0)),
            pl.BlockSpec((blk, D), lambda i: (i, 0)),
            pl.BlockSpec((blk, D), lambda i: (i, 0)),
            pl.BlockSpec((blk, D), lambda i: (i, 0)),
            pl.BlockSpec((blk, D), lambda i: (i, 0)),
            wspec((D, D)),
            wspec((D, D)),
            wspec((1, D)),
            wspec((D, D)),
            wspec((1, D)),
        ],
        out_specs=[
            pl.BlockSpec((blk, D), lambda i: (i, 0)),
            pl.BlockSpec((blk, P16), lambda i: (i, 0)),
        ],
        out_shape=[
            jax.ShapeDtypeStruct((N, D), jnp.float32),
            jax.ShapeDtypeStruct((N, P16), jnp.float32),
        ],
    )(h, pos16, pa0, pa1, pb0, pb1, nw1h, nw1m, nb1, nw2, nb2)


def kernel(h, pos, edge_index, msg_W1, msg_b1, msg_W2, msg_b2,
           coord_W1, coord_b1, coord_W2, coord_b2,
           node_W1, node_b1, node_W2, node_b2):
    src = edge_index[0]
    dst = edge_index[1]

    w1s = msg_W1[:D]
    w1d = msg_W1[D:2 * D]
    w1dist = msg_W1[2 * D].reshape(1, D)
    b1 = msg_b1.reshape(1, D)
    b2 = msg_b2.reshape(1, D)
    cb1 = coord_b1.reshape(1, D)
    cw2p = jnp.pad(coord_W2, ((0, 0), (0, 7)))
    cb2p = jnp.pad(coord_b2.reshape(1, 1), ((0, 0), (0, 7)))
    nw1h = node_W1[:D]
    nw1m = node_W1[D:]
    nb1 = node_b1.reshape(1, D)
    nb2 = node_b2.reshape(1, D)

    pos16 = jnp.pad(pos, ((0, 0), (0, P16 - 3)))
    z128 = jnp.zeros((N, D), jnp.float32)

    a, b = _preproj(h, w1s, w1d, b1)
    px = jnp.asarray(pos[:, 0])
    py = jnp.asarray(pos[:, 1])
    pz = jnp.asarray(pos[:, 2])
    ag, bg, geo = _sc_gather(a, b, px, py, pz, src, dst)
    msg, dcw = _edge_mlp(ag, bg, geo, w1dist, msg_W2, b2,
                         coord_W1, cb1, cw2p, cb2p)
    parts_a, parts_b = _sc_scatter(msg, dcw, dst, z128)
    h_new, pos16_new = _node_mlp(h, pos16, parts_a[0], parts_a[1],
                                 parts_b[0], parts_b[1],
                                 nw1h, nw1m, nb1, node_W2, nb2)
    return (h_new, pos16_new[:, :3])
